# baseline (speedup 1.0000x reference)
import functools

import jax
import jax.numpy as jnp
from jax import lax
from jax.experimental import pallas as pl
from jax.experimental.pallas import tpu as pltpu
from jax.experimental.pallas import tpu_sc as plsc

N = 10000
E = 320000
D = 128
HID = 32

NS = 16
LANE = 128
NPAD = 10240
SEG = NPAD // NS
PADIDX = NPAD - 1
ROWS = 2560
EPAD = ROWS * LANE
RPT = ROWS // NS

_mesh = plsc.VectorSubcoreMesh(core_axis_name="c", subcore_axis_name="s")


def _deg_body(src1, dst1, src2, dst2, o1, i1, o2, i2,
              sidx, didx, ones, zv, dego_sh, degi_sh):
    c = lax.axis_index("c")
    s = lax.axis_index("s")

    def fill(ref, val, n):
        def b(i, carry):
            ref[pl.ds(i * 16, 16)] = jnp.full((16,), val, jnp.float32)
            return carry
        lax.fori_loop(0, n // 16, b, 0)

    fill(ones, 1.0, LANE)
    fill(zv, 0.0, SEG)
    pltpu.sync_copy(zv, dego_sh.at[pl.ds(s * SEG, SEG)])
    pltpu.sync_copy(zv, degi_sh.at[pl.ds(s * SEG, SEG)])
    plsc.subcore_barrier()

    def run(srcr, dstr, outo, outi):
        pltpu.sync_copy(srcr.at[pl.ds(s * RPT, RPT)], sidx)
        pltpu.sync_copy(dstr.at[pl.ds(s * RPT, RPT)], didx)

        def b(j, carry):
            pltpu.sync_copy(ones, dego_sh.at[sidx.at[j]], add=True)
            pltpu.sync_copy(ones, degi_sh.at[didx.at[j]], add=True)
            return carry
        lax.fori_loop(0, RPT, b, 0)
        plsc.subcore_barrier()
        pltpu.sync_copy(dego_sh.at[pl.ds(s * SEG, SEG)],
                        outo.at[pl.ds(s * SEG, SEG)])
        pltpu.sync_copy(degi_sh.at[pl.ds(s * SEG, SEG)],
                        outi.at[pl.ds(s * SEG, SEG)])

    @pl.when(c == 0)
    def _():
        run(src1, dst1, o1, i1)

    @pl.when(c == 1)
    def _():
        run(src2, dst2, o2, i2)


_deg_kernel = functools.partial(
    pl.kernel, _deg_body, mesh=_mesh,
    out_type=[jax.ShapeDtypeStruct((NPAD,), jnp.float32)] * 4,
    scratch_types=[
        pltpu.VMEM((RPT, LANE), jnp.int32),
        pltpu.VMEM((RPT, LANE), jnp.int32),
        pltpu.VMEM((LANE,), jnp.float32),
        pltpu.VMEM((SEG,), jnp.float32),
        pltpu.VMEM_SHARED((NPAD,), jnp.float32),
        pltpu.VMEM_SHARED((NPAD,), jnp.float32),
    ],
)()


ZR = 64
DH = D // 2


NBUF = 4
NRND = RPT // NBUF


def _agg_body(xn1a, xn1b, xn2a, xn2b, src1, dst1, src2, dst2, xp,
              agg1a, agg1b, agg2a, agg2b, *scr):
    sidx, didx = scr[0], scr[1]
    bufs = scr[2:2 + NBUF]
    gse = scr[2 + NBUF:2 + 2 * NBUF]
    sse = scr[2 + 2 * NBUF:2 + 3 * NBUF]
    acc_sh = scr[2 + 3 * NBUF]
    c = lax.axis_index("c")
    s = lax.axis_index("s")
    zb = bufs[0]

    def run(xn, outr):
        def zfill(i, carry):
            zb[i // 4, pl.ds((i % 4) * 16, 16)] = jnp.zeros((16,), jnp.float32)
            return carry
        lax.fori_loop(0, LANE * (DH // 16), zfill, 0)

        def zc(j, carry):
            pltpu.sync_copy(zb, acc_sh.at[pl.ds(s * SEG + j * LANE, LANE)])
            return carry
        lax.fori_loop(0, SEG // LANE, zc, 0)
        plsc.subcore_barrier()

        for b in range(NBUF):
            pltpu.async_copy(xn.at[sidx.at[b]], bufs[b], gse[b])

        def rnd(k, carry):
            for b in range(NBUF):
                j = k * NBUF + b
                pltpu.make_async_copy(xn.at[sidx.at[j]], bufs[b],
                                      gse[b]).wait()
                pltpu.async_copy(bufs[b], acc_sh.at[didx.at[j]], sse[b],
                                 add=True)
            for b in range(NBUF):
                j = k * NBUF + b

                @pl.when(k < NRND - 1)
                def _(b=b, j=j):
                    pltpu.make_async_copy(bufs[b], acc_sh.at[didx.at[j]],
                                          sse[b]).wait()
                    pltpu.async_copy(xn.at[sidx.at[j + NBUF]], bufs[b],
                                     gse[b])
            return carry
        lax.fori_loop(0, NRND, rnd, 0)
        for b in range(NBUF):
            pltpu.make_async_copy(bufs[b],
                                  acc_sh.at[didx.at[RPT - NBUF + b]],
                                  sse[b]).wait()
        plsc.subcore_barrier()

        def wc(j, carry):
            pltpu.sync_copy(acc_sh.at[pl.ds(s * SEG + j * LANE, LANE)],
                            outr.at[pl.ds(s * SEG + j * LANE, LANE)])
            return carry
        lax.fori_loop(0, SEG // LANE, wc, 0)

    def runboth(srcr, dstr, xna, xnb, outa, outb):
        pltpu.sync_copy(srcr.at[pl.ds(s * RPT, RPT)], sidx)
        pltpu.sync_copy(dstr.at[pl.ds(s * RPT, RPT)], didx)
        for b in range(NBUF):
            pltpu.async_copy(xp.at[sidx.at[b]], bufs[b], gse[b])

        def rnd2(k, carry):
            for b in range(NBUF):
                j = k * NBUF + b
                pltpu.make_async_copy(xp.at[sidx.at[j]], bufs[b],
                                      gse[b]).wait()

                @pl.when(k < NRND - 1)
                def _(b=b, j=j):
                    pltpu.async_copy(xp.at[sidx.at[j + NBUF]], bufs[b],
                                     gse[b])
            return carry
        lax.fori_loop(0, NRND, rnd2, 0)

    @pl.when(c == 0)
    def _():
        runboth(src1, dst1, xn1a, xn1b, agg1a, agg1b)

    @pl.when(c == 1)
    def _():
        runboth(src2, dst2, xn2a, xn2b, agg2a, agg2b)


_agg_kernel = functools.partial(
    pl.kernel, _agg_body, mesh=_mesh,
    out_type=[jax.ShapeDtypeStruct((NPAD, DH), jnp.float32)] * 4,
    scratch_types=(
        [pltpu.VMEM((RPT, LANE), jnp.int32) for _ in range(2)]
        + [pltpu.VMEM((LANE, D), jnp.float32) for _ in range(NBUF)]
        + [pltpu.SemaphoreType.DMA for _ in range(2 * NBUF)]
        + [pltpu.VMEM_SHARED((NPAD, DH), jnp.float32)]
    ),
    compiler_params=pltpu.CompilerParams(use_tc_tiling_on_sc=False),
)()


BN = 1024


def _scale_body(x_ref, dego_ref, xn1a_ref, xn1b_ref, xn2a_ref, xn2b_ref):
    sc = lax.rsqrt(jnp.maximum(dego_ref[...], 1.0))
    xv = x_ref[...]
    xn1 = xv * sc[:, 0:1]
    xn2 = xv * sc[:, 1:2]
    xn1a_ref[...] = xn1[:, :DH]
    xn1b_ref[...] = xn1[:, DH:]
    xn2a_ref[...] = xn2[:, :DH]
    xn2b_ref[...] = xn2[:, DH:]


def _scale_call(x_pad, dego):
    return pl.pallas_call(
        _scale_body,
        grid=(NPAD // BN,),
        in_specs=[
            pl.BlockSpec((BN, D), lambda i: (i, 0)),
            pl.BlockSpec((BN, 2), lambda i: (i, 0)),
        ],
        out_specs=[pl.BlockSpec((BN, DH), lambda i: (i, 0))] * 4,
        out_shape=[jax.ShapeDtypeStruct((NPAD, DH), jnp.float32)] * 4,
    )(x_pad, dego)


BC = 1000
NB = N // BC


def _post_body(a1a_ref, a1b_ref, a2a_ref, a2b_ref, degi_ref, wg_ref, bg_ref,
               w1_ref, b1_ref, w2_ref, z_ref, part_ref):
    si = lax.rsqrt(jnp.maximum(degi_ref[...], 1.0))
    wg = wg_ref[...]
    bg = bg_ref[...]
    a1 = jnp.concatenate([a1a_ref[...], a1b_ref[...]], axis=1)
    a2 = jnp.concatenate([a2a_ref[...], a2b_ref[...]], axis=1)
    e1 = jnp.maximum(jnp.dot(a1 * si[:, 0:1], wg,
                             preferred_element_type=jnp.float32) + bg, 0.0)
    e2 = jnp.maximum(jnp.dot(a2 * si[:, 1:2], wg,
                             preferred_element_type=jnp.float32) + bg, 0.0)
    z_ref[...] = jnp.stack([e1, e2], axis=1)
    w1 = w1_ref[...]
    b1 = b1_ref[...]
    w2 = w2_ref[...]
    p1 = jnp.sum(jnp.dot(jnp.tanh(jnp.dot(e1, w1,
                                          preferred_element_type=jnp.float32)
                                  + b1), w2,
                         preferred_element_type=jnp.float32))
    p2 = jnp.sum(jnp.dot(jnp.tanh(jnp.dot(e2, w1,
                                          preferred_element_type=jnp.float32)
                                  + b1), w2,
                         preferred_element_type=jnp.float32))
    part_ref[...] = jnp.stack([p1, p2]).reshape(1, 1, 2)


def _post_call(agg1a, agg1b, agg2a, agg2b, degi, Wg, bg2, W1, b12, W2):
    return pl.pallas_call(
        _post_body,
        grid=(NB,),
        in_specs=[
            pl.BlockSpec((BC, DH), lambda i: (i, 0)),
            pl.BlockSpec((BC, DH), lambda i: (i, 0)),
            pl.BlockSpec((BC, DH), lambda i: (i, 0)),
            pl.BlockSpec((BC, DH), lambda i: (i, 0)),
            pl.BlockSpec((BC, 2), lambda i: (i, 0)),
            pl.BlockSpec((D, D), lambda i: (0, 0)),
            pl.BlockSpec((1, D), lambda i: (0, 0)),
            pl.BlockSpec((D, HID), lambda i: (0, 0)),
            pl.BlockSpec((1, HID), lambda i: (0, 0)),
            pl.BlockSpec((HID, 1), lambda i: (0, 0)),
        ],
        out_specs=[
            pl.BlockSpec((BC, 2, D), lambda i: (i, 0, 0)),
            pl.BlockSpec((1, 1, 2), lambda i: (i, 0, 0)),
        ],
        out_shape=[
            jax.ShapeDtypeStruct((N, 2, D), jnp.float32),
            jax.ShapeDtypeStruct((NB, 1, 2), jnp.float32),
        ],
    )(agg1a, agg1b, agg2a, agg2b, degi, Wg, bg2, W1, b12, W2)


def _comb_body(part_ref, z_ref, out_ref):
    w = jnp.sum(part_ref[...], axis=0) * (1.0 / N)
    m = jnp.max(w)
    ew = jnp.exp(w - m)
    beta = ew / jnp.sum(ew)
    zz = z_ref[...]
    out_ref[...] = (zz[:, 0, :] * beta[0:1, 0:1]
                    + zz[:, 1, :] * beta[0:1, 1:2])


def _comb_call(part, z):
    return pl.pallas_call(
        _comb_body,
        grid=(NB,),
        in_specs=[
            pl.BlockSpec((NB, 1, 2), lambda i: (0, 0, 0)),
            pl.BlockSpec((BC, 2, D), lambda i: (i, 0, 0)),
        ],
        out_specs=pl.BlockSpec((BC, D), lambda i: (i, 0)),
        out_shape=jax.ShapeDtypeStruct((N, D), jnp.float32),
    )(part, z)


def _pad_idx(a):
    a = jnp.pad(a, (0, EPAD - E), constant_values=PADIDX)
    return a.reshape(ROWS, LANE)


def kernel(x, edge_index1, edge_index2, Wg, bg, W1, b1, W2):
    src1 = _pad_idx(edge_index1[0])
    dst1 = _pad_idx(edge_index1[1])
    src2 = _pad_idx(edge_index2[0])
    dst2 = _pad_idx(edge_index2[1])

    do1, di1, do2, di2 = _deg_kernel(src1, dst1, src2, dst2)
    dego = jnp.stack([do1, do2], axis=1)
    degi = jnp.stack([di1, di2], axis=1)

    x_pad = jnp.pad(x, ((0, NPAD - N), (0, 0)))
    xn1a, xn1b, xn2a, xn2b = _scale_call(x_pad, dego)

    agg1a, agg1b, agg2a, agg2b = _agg_kernel(
        xn1a, xn1b, xn2a, xn2b, src1, dst1, src2, dst2, x_pad)

    z, part = _post_call(agg1a, agg1b, agg2a, agg2b, degi,
                         Wg, bg.reshape(1, D), W1, b1.reshape(1, HID), W2)
    out = _comb_call(part, z)
    return (out, z)


# speedup vs baseline: 1.5321x; 1.5321x over previous
import functools

import jax
import jax.numpy as jnp
from jax import lax
from jax.experimental import pallas as pl
from jax.experimental.pallas import tpu as pltpu
from jax.experimental.pallas import tpu_sc as plsc

N = 10000
E = 320000
D = 128
HID = 32

NS = 16
LANE = 128
NPAD = 10240
SEG = NPAD // NS
PADIDX = NPAD - 1
ROWS = 2560
EPAD = ROWS * LANE
RPT = ROWS // NS

_mesh = plsc.VectorSubcoreMesh(core_axis_name="c", subcore_axis_name="s")


def _deg_body(src1, dst1, src2, dst2, o1, i1, o2, i2,
              sidx, didx, ones, zv, dego_sh, degi_sh):
    c = lax.axis_index("c")
    s = lax.axis_index("s")

    def fill(ref, val, n):
        def b(i, carry):
            ref[pl.ds(i * 16, 16)] = jnp.full((16,), val, jnp.float32)
            return carry
        lax.fori_loop(0, n // 16, b, 0)

    fill(ones, 1.0, LANE)
    fill(zv, 0.0, SEG)
    pltpu.sync_copy(zv, dego_sh.at[pl.ds(s * SEG, SEG)])
    pltpu.sync_copy(zv, degi_sh.at[pl.ds(s * SEG, SEG)])
    plsc.subcore_barrier()

    def run(srcr, dstr, outo, outi):
        pltpu.sync_copy(srcr.at[pl.ds(s * RPT, RPT)], sidx)
        pltpu.sync_copy(dstr.at[pl.ds(s * RPT, RPT)], didx)

        def b(j, carry):
            pltpu.sync_copy(ones, dego_sh.at[sidx.at[j]], add=True)
            pltpu.sync_copy(ones, degi_sh.at[didx.at[j]], add=True)
            return carry
        lax.fori_loop(0, RPT, b, 0)
        plsc.subcore_barrier()
        pltpu.sync_copy(dego_sh.at[pl.ds(s * SEG, SEG)],
                        outo.at[pl.ds(s * SEG, SEG)])
        pltpu.sync_copy(degi_sh.at[pl.ds(s * SEG, SEG)],
                        outi.at[pl.ds(s * SEG, SEG)])

    @pl.when(c == 0)
    def _():
        run(src1, dst1, o1, i1)

    @pl.when(c == 1)
    def _():
        run(src2, dst2, o2, i2)


_deg_kernel = functools.partial(
    pl.kernel, _deg_body, mesh=_mesh,
    out_type=[jax.ShapeDtypeStruct((NPAD,), jnp.float32)] * 4,
    scratch_types=[
        pltpu.VMEM((RPT, LANE), jnp.int32),
        pltpu.VMEM((RPT, LANE), jnp.int32),
        pltpu.VMEM((LANE,), jnp.float32),
        pltpu.VMEM((SEG,), jnp.float32),
        pltpu.VMEM_SHARED((NPAD,), jnp.float32),
        pltpu.VMEM_SHARED((NPAD,), jnp.float32),
    ],
)()


NQ = 4
DQ = D // NQ
NBUF = 4
NRND = RPT // NBUF


def _agg_body(*refs):
    xnq = (refs[0:NQ], refs[NQ:2 * NQ])
    src1, dst1, src2, dst2 = refs[2 * NQ:2 * NQ + 4]
    outq = (refs[2 * NQ + 4:3 * NQ + 4], refs[3 * NQ + 4:4 * NQ + 4])
    scr = refs[4 * NQ + 4:]
    sidx, didx = scr[0], scr[1]
    bufs = scr[2:2 + NBUF]
    gse = scr[2 + NBUF:2 + 2 * NBUF]
    sse = scr[2 + 2 * NBUF:2 + 3 * NBUF]
    xq_sh = scr[2 + 3 * NBUF]
    acc_sh = scr[3 + 3 * NBUF]
    c = lax.axis_index("c")
    s = lax.axis_index("s")
    zb = bufs[0]

    def qpass(xn, outr):
        pltpu.sync_copy(xn.at[pl.ds(s * SEG, SEG)],
                        xq_sh.at[pl.ds(s * SEG, SEG)])

        def zfill(i, carry):
            zb[i // 2, pl.ds((i % 2) * 16, 16)] = jnp.zeros((16,), jnp.float32)
            return carry
        lax.fori_loop(0, LANE * (DQ // 16), zfill, 0)

        def zc(j, carry):
            pltpu.sync_copy(zb, acc_sh.at[pl.ds(s * SEG + j * LANE, LANE)])
            return carry
        lax.fori_loop(0, SEG // LANE, zc, 0)
        plsc.subcore_barrier()

        for b in range(NBUF):
            pltpu.async_copy(xq_sh.at[sidx.at[b]], bufs[b], gse[b])

        def rnd(k, carry):
            for b in range(NBUF):
                j = k * NBUF + b
                pltpu.make_async_copy(xq_sh.at[sidx.at[j]], bufs[b],
                                      gse[b]).wait()
                pltpu.async_copy(bufs[b], acc_sh.at[didx.at[j]], sse[b],
                                 add=True)
            for b in range(NBUF):
                j = k * NBUF + b

                @pl.when(k < NRND - 1)
                def _(b=b, j=j):
                    pltpu.make_async_copy(bufs[b], acc_sh.at[didx.at[j]],
                                          sse[b]).wait()
                    pltpu.async_copy(xq_sh.at[sidx.at[j + NBUF]], bufs[b],
                                     gse[b])
            return carry
        lax.fori_loop(0, NRND, rnd, 0)
        for b in range(NBUF):
            pltpu.make_async_copy(bufs[b],
                                  acc_sh.at[didx.at[RPT - NBUF + b]],
                                  sse[b]).wait()
        plsc.subcore_barrier()

        def wc(j, carry):
            pltpu.sync_copy(acc_sh.at[pl.ds(s * SEG + j * LANE, LANE)],
                            outr.at[pl.ds(s * SEG + j * LANE, LANE)])
            return carry
        lax.fori_loop(0, SEG // LANE, wc, 0)

    def runboth(srcr, dstr, m):
        pltpu.sync_copy(srcr.at[pl.ds(s * RPT, RPT)], sidx)
        pltpu.sync_copy(dstr.at[pl.ds(s * RPT, RPT)], didx)
        for q in range(NQ):
            qpass(xnq[m][q], outq[m][q])

    @pl.when(c == 0)
    def _():
        runboth(src1, dst1, 0)

    @pl.when(c == 1)
    def _():
        runboth(src2, dst2, 1)


_agg_kernel = functools.partial(
    pl.kernel, _agg_body, mesh=_mesh,
    out_type=[jax.ShapeDtypeStruct((NPAD, DQ), jnp.float32)] * (2 * NQ),
    scratch_types=(
        [pltpu.VMEM((RPT, LANE), jnp.int32) for _ in range(2)]
        + [pltpu.VMEM((LANE, DQ), jnp.float32) for _ in range(NBUF)]
        + [pltpu.SemaphoreType.DMA for _ in range(2 * NBUF)]
        + [pltpu.VMEM_SHARED((NPAD, DQ), jnp.float32) for _ in range(2)]
    ),
    compiler_params=pltpu.CompilerParams(use_tc_tiling_on_sc=False),
)()


BN = 1024


def _scale_body(x_ref, dego_ref, *outs):
    sc = lax.rsqrt(jnp.maximum(dego_ref[...], 1.0))
    xv = x_ref[...]
    xn1 = xv * sc[:, 0:1]
    xn2 = xv * sc[:, 1:2]
    for q in range(NQ):
        outs[q][...] = xn1[:, q * DQ:(q + 1) * DQ]
        outs[NQ + q][...] = xn2[:, q * DQ:(q + 1) * DQ]


def _scale_call(x_pad, dego):
    return pl.pallas_call(
        _scale_body,
        grid=(NPAD // BN,),
        in_specs=[
            pl.BlockSpec((BN, D), lambda i: (i, 0)),
            pl.BlockSpec((BN, 2), lambda i: (i, 0)),
        ],
        out_specs=[pl.BlockSpec((BN, DQ), lambda i: (i, 0))] * (2 * NQ),
        out_shape=[jax.ShapeDtypeStruct((NPAD, DQ), jnp.float32)] * (2 * NQ),
    )(x_pad, dego)


BC = 1000
NB = N // BC


def _post_body(*refs):
    aq = refs[0:2 * NQ]
    degi_ref, wg_ref, bg_ref, w1_ref, b1_ref, w2_ref = refs[2 * NQ:2 * NQ + 6]
    z_ref, part_ref = refs[2 * NQ + 6], refs[2 * NQ + 7]
    si = lax.rsqrt(jnp.maximum(degi_ref[...], 1.0))
    wg = wg_ref[...]
    bg = bg_ref[...]
    a1 = jnp.concatenate([aq[q][...] for q in range(NQ)], axis=1)
    a2 = jnp.concatenate([aq[NQ + q][...] for q in range(NQ)], axis=1)
    e1 = jnp.maximum(jnp.dot(a1 * si[:, 0:1], wg,
                             preferred_element_type=jnp.float32) + bg, 0.0)
    e2 = jnp.maximum(jnp.dot(a2 * si[:, 1:2], wg,
                             preferred_element_type=jnp.float32) + bg, 0.0)
    z_ref[...] = jnp.stack([e1, e2], axis=1)
    w1 = w1_ref[...]
    b1 = b1_ref[...]
    w2 = w2_ref[...]
    p1 = jnp.sum(jnp.dot(jnp.tanh(jnp.dot(e1, w1,
                                          preferred_element_type=jnp.float32)
                                  + b1), w2,
                         preferred_element_type=jnp.float32))
    p2 = jnp.sum(jnp.dot(jnp.tanh(jnp.dot(e2, w1,
                                          preferred_element_type=jnp.float32)
                                  + b1), w2,
                         preferred_element_type=jnp.float32))
    part_ref[...] = jnp.stack([p1, p2]).reshape(1, 1, 2)


def _post_call(aggq, degi, Wg, bg2, W1, b12, W2):
    return pl.pallas_call(
        _post_body,
        grid=(NB,),
        in_specs=[pl.BlockSpec((BC, DQ), lambda i: (i, 0))] * (2 * NQ) + [
            pl.BlockSpec((BC, 2), lambda i: (i, 0)),
            pl.BlockSpec((D, D), lambda i: (0, 0)),
            pl.BlockSpec((1, D), lambda i: (0, 0)),
            pl.BlockSpec((D, HID), lambda i: (0, 0)),
            pl.BlockSpec((1, HID), lambda i: (0, 0)),
            pl.BlockSpec((HID, 1), lambda i: (0, 0)),
        ],
        out_specs=[
            pl.BlockSpec((BC, 2, D), lambda i: (i, 0, 0)),
            pl.BlockSpec((1, 1, 2), lambda i: (i, 0, 0)),
        ],
        out_shape=[
            jax.ShapeDtypeStruct((N, 2, D), jnp.float32),
            jax.ShapeDtypeStruct((NB, 1, 2), jnp.float32),
        ],
    )(*aggq, degi, Wg, bg2, W1, b12, W2)


def _comb_body(part_ref, z_ref, out_ref):
    w = jnp.sum(part_ref[...], axis=0) * (1.0 / N)
    m = jnp.max(w)
    ew = jnp.exp(w - m)
    beta = ew / jnp.sum(ew)
    zz = z_ref[...]
    out_ref[...] = (zz[:, 0, :] * beta[0:1, 0:1]
                    + zz[:, 1, :] * beta[0:1, 1:2])


def _comb_call(part, z):
    return pl.pallas_call(
        _comb_body,
        grid=(NB,),
        in_specs=[
            pl.BlockSpec((NB, 1, 2), lambda i: (0, 0, 0)),
            pl.BlockSpec((BC, 2, D), lambda i: (i, 0, 0)),
        ],
        out_specs=pl.BlockSpec((BC, D), lambda i: (i, 0)),
        out_shape=jax.ShapeDtypeStruct((N, D), jnp.float32),
    )(part, z)


def _pad_idx(a):
    a = jnp.pad(a, (0, EPAD - E), constant_values=PADIDX)
    return a.reshape(ROWS, LANE)


def kernel(x, edge_index1, edge_index2, Wg, bg, W1, b1, W2):
    src1 = _pad_idx(edge_index1[0])
    dst1 = _pad_idx(edge_index1[1])
    src2 = _pad_idx(edge_index2[0])
    dst2 = _pad_idx(edge_index2[1])

    do1, di1, do2, di2 = _deg_kernel(src1, dst1, src2, dst2)
    dego = jnp.stack([do1, do2], axis=1)
    degi = jnp.stack([di1, di2], axis=1)

    x_pad = jnp.pad(x, ((0, NPAD - N), (0, 0)))
    xnq = _scale_call(x_pad, dego)

    aggq = _agg_kernel(*xnq, src1, dst1, src2, dst2)

    z, part = _post_call(aggq, degi, Wg, bg.reshape(1, D),
                         W1, b1.reshape(1, HID), W2)
    out = _comb_call(part, z)
    return (out, z)


# speedup vs baseline: 1.5628x; 1.0201x over previous
import functools

import jax
import jax.numpy as jnp
from jax import lax
from jax.experimental import pallas as pl
from jax.experimental.pallas import tpu as pltpu
from jax.experimental.pallas import tpu_sc as plsc

N = 10000
E = 320000
D = 128
HID = 32

NS = 16
LANE = 128
NPAD = 10240
SEG = NPAD // NS
PADIDX = NPAD - 1
ROWS = 2560
EPAD = ROWS * LANE
RPT = ROWS // NS

_mesh = plsc.VectorSubcoreMesh(core_axis_name="c", subcore_axis_name="s")


def _deg_body(src1, dst1, src2, dst2, o1, i1, o2, i2,
              sidx, didx, ones, zv, dego_sh, degi_sh):
    c = lax.axis_index("c")
    s = lax.axis_index("s")

    def fill(ref, val, n):
        def b(i, carry):
            ref[pl.ds(i * 16, 16)] = jnp.full((16,), val, jnp.float32)
            return carry
        lax.fori_loop(0, n // 16, b, 0)

    fill(ones, 1.0, LANE)
    fill(zv, 0.0, SEG)
    pltpu.sync_copy(zv, dego_sh.at[pl.ds(s * SEG, SEG)])
    pltpu.sync_copy(zv, degi_sh.at[pl.ds(s * SEG, SEG)])
    plsc.subcore_barrier()

    def run(srcr, dstr, outo, outi):
        pltpu.sync_copy(srcr.at[pl.ds(s * RPT, RPT)], sidx)
        pltpu.sync_copy(dstr.at[pl.ds(s * RPT, RPT)], didx)

        def b(j, carry):
            pltpu.sync_copy(ones, dego_sh.at[sidx.at[j]], add=True)
            pltpu.sync_copy(ones, degi_sh.at[didx.at[j]], add=True)
            return carry
        lax.fori_loop(0, RPT, b, 0)
        plsc.subcore_barrier()
        pltpu.sync_copy(dego_sh.at[pl.ds(s * SEG, SEG)],
                        outo.at[pl.ds(s * SEG, SEG)])
        pltpu.sync_copy(degi_sh.at[pl.ds(s * SEG, SEG)],
                        outi.at[pl.ds(s * SEG, SEG)])

    @pl.when(c == 0)
    def _():
        run(src1, dst1, o1, i1)

    @pl.when(c == 1)
    def _():
        run(src2, dst2, o2, i2)


_deg_kernel = functools.partial(
    pl.kernel, _deg_body, mesh=_mesh,
    out_type=[jax.ShapeDtypeStruct((NPAD,), jnp.float32)] * 4,
    scratch_types=[
        pltpu.VMEM((RPT, LANE), jnp.int32),
        pltpu.VMEM((RPT, LANE), jnp.int32),
        pltpu.VMEM((LANE,), jnp.float32),
        pltpu.VMEM((SEG,), jnp.float32),
        pltpu.VMEM_SHARED((NPAD,), jnp.float32),
        pltpu.VMEM_SHARED((NPAD,), jnp.float32),
    ],
)()


NQ = 4
DQ = D // NQ
NBUF = 8
NRND = RPT // NBUF


def _agg_body(*refs):
    xnq = (refs[0:NQ], refs[NQ:2 * NQ])
    src1, dst1, src2, dst2 = refs[2 * NQ:2 * NQ + 4]
    outq = (refs[2 * NQ + 4:3 * NQ + 4], refs[3 * NQ + 4:4 * NQ + 4])
    scr = refs[4 * NQ + 4:]
    sidx, didx = scr[0], scr[1]
    bufs = scr[2:2 + NBUF]
    gse = scr[2 + NBUF:2 + 2 * NBUF]
    sse = scr[2 + 2 * NBUF:2 + 3 * NBUF]
    xq_sh = scr[2 + 3 * NBUF]
    acc_sh = scr[3 + 3 * NBUF]
    c = lax.axis_index("c")
    s = lax.axis_index("s")
    zb = bufs[0]

    def qpass(xn, outr):
        pltpu.sync_copy(xn.at[pl.ds(s * SEG, SEG)],
                        xq_sh.at[pl.ds(s * SEG, SEG)])

        def zfill(i, carry):
            zb[i // 2, pl.ds((i % 2) * 16, 16)] = jnp.zeros((16,), jnp.float32)
            return carry
        lax.fori_loop(0, LANE * (DQ // 16), zfill, 0)

        def zc(j, carry):
            pltpu.sync_copy(zb, acc_sh.at[pl.ds(s * SEG + j * LANE, LANE)])
            return carry
        lax.fori_loop(0, SEG // LANE, zc, 0)
        plsc.subcore_barrier()

        for b in range(NBUF):
            pltpu.async_copy(xq_sh.at[sidx.at[b]], bufs[b], gse[b])

        def rnd(k, carry):
            for b in range(NBUF):
                j = k * NBUF + b
                pltpu.make_async_copy(xq_sh.at[sidx.at[j]], bufs[b],
                                      gse[b]).wait()
                pltpu.async_copy(bufs[b], acc_sh.at[didx.at[j]], sse[b],
                                 add=True)
            for b in range(NBUF):
                j = k * NBUF + b

                @pl.when(k < NRND - 1)
                def _(b=b, j=j):
                    pltpu.make_async_copy(bufs[b], acc_sh.at[didx.at[j]],
                                          sse[b]).wait()
                    pltpu.async_copy(xq_sh.at[sidx.at[j + NBUF]], bufs[b],
                                     gse[b])
            return carry
        lax.fori_loop(0, NRND, rnd, 0)
        for b in range(NBUF):
            pltpu.make_async_copy(bufs[b],
                                  acc_sh.at[didx.at[RPT - NBUF + b]],
                                  sse[b]).wait()
        plsc.subcore_barrier()

        def wc(j, carry):
            pltpu.sync_copy(acc_sh.at[pl.ds(s * SEG + j * LANE, LANE)],
                            outr.at[pl.ds(s * SEG + j * LANE, LANE)])
            return carry
        lax.fori_loop(0, SEG // LANE, wc, 0)

    def runboth(srcr, dstr, m):
        pltpu.sync_copy(srcr.at[pl.ds(s * RPT, RPT)], sidx)
        pltpu.sync_copy(dstr.at[pl.ds(s * RPT, RPT)], didx)
        for q in range(NQ):
            qpass(xnq[m][q], outq[m][q])

    @pl.when(c == 0)
    def _():
        runboth(src1, dst1, 0)

    @pl.when(c == 1)
    def _():
        runboth(src2, dst2, 1)


_agg_kernel = functools.partial(
    pl.kernel, _agg_body, mesh=_mesh,
    out_type=[jax.ShapeDtypeStruct((NPAD, DQ), jnp.float32)] * (2 * NQ),
    scratch_types=(
        [pltpu.VMEM((RPT, LANE), jnp.int32) for _ in range(2)]
        + [pltpu.VMEM((LANE, DQ), jnp.float32) for _ in range(NBUF)]
        + [pltpu.SemaphoreType.DMA for _ in range(2 * NBUF)]
        + [pltpu.VMEM_SHARED((NPAD, DQ), jnp.float32) for _ in range(2)]
    ),
    compiler_params=pltpu.CompilerParams(use_tc_tiling_on_sc=False),
)()


BN = 1024


def _scale_body(x_ref, dego_ref, *outs):
    sc = lax.rsqrt(jnp.maximum(dego_ref[...], 1.0))
    xv = x_ref[...]
    xn1 = xv * sc[:, 0:1]
    xn2 = xv * sc[:, 1:2]
    for q in range(NQ):
        outs[q][...] = xn1[:, q * DQ:(q + 1) * DQ]
        outs[NQ + q][...] = xn2[:, q * DQ:(q + 1) * DQ]


def _scale_call(x_pad, dego):
    return pl.pallas_call(
        _scale_body,
        grid=(NPAD // BN,),
        in_specs=[
            pl.BlockSpec((BN, D), lambda i: (i, 0)),
            pl.BlockSpec((BN, 2), lambda i: (i, 0)),
        ],
        out_specs=[pl.BlockSpec((BN, DQ), lambda i: (i, 0))] * (2 * NQ),
        out_shape=[jax.ShapeDtypeStruct((NPAD, DQ), jnp.float32)] * (2 * NQ),
    )(x_pad, dego)


BC = 1000
NB = N // BC


def _post_body(*refs):
    aq = refs[0:2 * NQ]
    degi_ref, wg_ref, bg_ref, w1_ref, b1_ref, w2_ref = refs[2 * NQ:2 * NQ + 6]
    z_ref, part_ref = refs[2 * NQ + 6], refs[2 * NQ + 7]
    si = lax.rsqrt(jnp.maximum(degi_ref[...], 1.0))
    wg = wg_ref[...]
    bg = bg_ref[...]
    a1 = jnp.concatenate([aq[q][...] for q in range(NQ)], axis=1)
    a2 = jnp.concatenate([aq[NQ + q][...] for q in range(NQ)], axis=1)
    e1 = jnp.maximum(jnp.dot(a1 * si[:, 0:1], wg,
                             preferred_element_type=jnp.float32) + bg, 0.0)
    e2 = jnp.maximum(jnp.dot(a2 * si[:, 1:2], wg,
                             preferred_element_type=jnp.float32) + bg, 0.0)
    z_ref[...] = jnp.stack([e1, e2], axis=1)
    w1 = w1_ref[...]
    b1 = b1_ref[...]
    w2 = w2_ref[...]
    p1 = jnp.sum(jnp.dot(jnp.tanh(jnp.dot(e1, w1,
                                          preferred_element_type=jnp.float32)
                                  + b1), w2,
                         preferred_element_type=jnp.float32))
    p2 = jnp.sum(jnp.dot(jnp.tanh(jnp.dot(e2, w1,
                                          preferred_element_type=jnp.float32)
                                  + b1), w2,
                         preferred_element_type=jnp.float32))
    part_ref[...] = jnp.stack([p1, p2]).reshape(1, 1, 2)


def _post_call(aggq, degi, Wg, bg2, W1, b12, W2):
    return pl.pallas_call(
        _post_body,
        grid=(NB,),
        in_specs=[pl.BlockSpec((BC, DQ), lambda i: (i, 0))] * (2 * NQ) + [
            pl.BlockSpec((BC, 2), lambda i: (i, 0)),
            pl.BlockSpec((D, D), lambda i: (0, 0)),
            pl.BlockSpec((1, D), lambda i: (0, 0)),
            pl.BlockSpec((D, HID), lambda i: (0, 0)),
            pl.BlockSpec((1, HID), lambda i: (0, 0)),
            pl.BlockSpec((HID, 1), lambda i: (0, 0)),
        ],
        out_specs=[
            pl.BlockSpec((BC, 2, D), lambda i: (i, 0, 0)),
            pl.BlockSpec((1, 1, 2), lambda i: (i, 0, 0)),
        ],
        out_shape=[
            jax.ShapeDtypeStruct((N, 2, D), jnp.float32),
            jax.ShapeDtypeStruct((NB, 1, 2), jnp.float32),
        ],
    )(*aggq, degi, Wg, bg2, W1, b12, W2)


def _comb_body(part_ref, z_ref, out_ref):
    w = jnp.sum(part_ref[...], axis=0) * (1.0 / N)
    m = jnp.max(w)
    ew = jnp.exp(w - m)
    beta = ew / jnp.sum(ew)
    zz = z_ref[...]
    out_ref[...] = (zz[:, 0, :] * beta[0:1, 0:1]
                    + zz[:, 1, :] * beta[0:1, 1:2])


def _comb_call(part, z):
    return pl.pallas_call(
        _comb_body,
        grid=(NB,),
        in_specs=[
            pl.BlockSpec((NB, 1, 2), lambda i: (0, 0, 0)),
            pl.BlockSpec((BC, 2, D), lambda i: (i, 0, 0)),
        ],
        out_specs=pl.BlockSpec((BC, D), lambda i: (i, 0)),
        out_shape=jax.ShapeDtypeStruct((N, D), jnp.float32),
    )(part, z)


def _pad_idx(a):
    a = jnp.pad(a, (0, EPAD - E), constant_values=PADIDX)
    return a.reshape(ROWS, LANE)


def kernel(x, edge_index1, edge_index2, Wg, bg, W1, b1, W2):
    src1 = _pad_idx(edge_index1[0])
    dst1 = _pad_idx(edge_index1[1])
    src2 = _pad_idx(edge_index2[0])
    dst2 = _pad_idx(edge_index2[1])

    do1, di1, do2, di2 = _deg_kernel(src1, dst1, src2, dst2)
    dego = jnp.stack([do1, do2], axis=1)
    degi = jnp.stack([di1, di2], axis=1)

    x_pad = jnp.pad(x, ((0, NPAD - N), (0, 0)))
    xnq = _scale_call(x_pad, dego)

    aggq = _agg_kernel(*xnq, src1, dst1, src2, dst2)

    z, part = _post_call(aggq, degi, Wg, bg.reshape(1, D),
                         W1, b1.reshape(1, HID), W2)
    out = _comb_call(part, z)
    return (out, z)


# speedup vs baseline: 1.6095x; 1.0298x over previous
import functools

import jax
import jax.numpy as jnp
from jax import lax
from jax.experimental import pallas as pl
from jax.experimental.pallas import tpu as pltpu
from jax.experimental.pallas import tpu_sc as plsc

N = 10000
E = 320000
D = 128
HID = 32

NS = 16
LANE = 128
NPAD = 10240
SEG = NPAD // NS
PADIDX = NPAD - 1
ROWS = 2560
EPAD = ROWS * LANE
RPT = ROWS // NS

_mesh = plsc.VectorSubcoreMesh(core_axis_name="c", subcore_axis_name="s")


DEGQ = 8


def _deg_body(src1, dst1, src2, dst2, o1, i1, o2, i2,
              sidx, didx, ones, zv, se0, se1, dego_sh, degi_sh):
    c = lax.axis_index("c")
    s = lax.axis_index("s")

    def fill(ref, val, n):
        def b(i, carry):
            ref[pl.ds(i * 16, 16)] = jnp.full((16,), val, jnp.float32)
            return carry
        lax.fori_loop(0, n // 16, b, 0)

    fill(ones, 1.0, LANE)
    fill(zv, 0.0, SEG)
    pltpu.sync_copy(zv, dego_sh.at[pl.ds(s * SEG, SEG)])
    pltpu.sync_copy(zv, degi_sh.at[pl.ds(s * SEG, SEG)])
    plsc.subcore_barrier()

    def run(srcr, dstr, outo, outi):
        pltpu.sync_copy(srcr.at[pl.ds(s * RPT, RPT)], sidx)
        pltpu.sync_copy(dstr.at[pl.ds(s * RPT, RPT)], didx)

        def b(j, carry):
            @pl.when(j >= DEGQ)
            def _():
                pltpu.make_async_copy(ones, dego_sh.at[sidx.at[j - DEGQ]],
                                      se0).wait()
                pltpu.make_async_copy(ones, degi_sh.at[didx.at[j - DEGQ]],
                                      se1).wait()
            pltpu.async_copy(ones, dego_sh.at[sidx.at[j]], se0, add=True)
            pltpu.async_copy(ones, degi_sh.at[didx.at[j]], se1, add=True)
            return carry
        lax.fori_loop(0, RPT, b, 0)
        for j in range(RPT - DEGQ, RPT):
            pltpu.make_async_copy(ones, dego_sh.at[sidx.at[j]], se0).wait()
            pltpu.make_async_copy(ones, degi_sh.at[didx.at[j]], se1).wait()
        plsc.subcore_barrier()
        pltpu.sync_copy(dego_sh.at[pl.ds(s * SEG, SEG)],
                        outo.at[pl.ds(s * SEG, SEG)])
        pltpu.sync_copy(degi_sh.at[pl.ds(s * SEG, SEG)],
                        outi.at[pl.ds(s * SEG, SEG)])

    @pl.when(c == 0)
    def _():
        run(src1, dst1, o1, i1)

    @pl.when(c == 1)
    def _():
        run(src2, dst2, o2, i2)


_deg_kernel = functools.partial(
    pl.kernel, _deg_body, mesh=_mesh,
    out_type=[jax.ShapeDtypeStruct((NPAD,), jnp.float32)] * 4,
    scratch_types=[
        pltpu.VMEM((RPT, LANE), jnp.int32),
        pltpu.VMEM((RPT, LANE), jnp.int32),
        pltpu.VMEM((LANE,), jnp.float32),
        pltpu.VMEM((SEG,), jnp.float32),
        pltpu.SemaphoreType.DMA,
        pltpu.SemaphoreType.DMA,
        pltpu.VMEM_SHARED((NPAD,), jnp.float32),
        pltpu.VMEM_SHARED((NPAD,), jnp.float32),
    ],
)()


NQ = 4
DQ = D // NQ
NBUF = 8
NRND = RPT // NBUF


def _agg_body(*refs):
    xnq = (refs[0:NQ], refs[NQ:2 * NQ])
    src1, dst1, src2, dst2 = refs[2 * NQ:2 * NQ + 4]
    outq = (refs[2 * NQ + 4:3 * NQ + 4], refs[3 * NQ + 4:4 * NQ + 4])
    scr = refs[4 * NQ + 4:]
    sidx, didx = scr[0], scr[1]
    bufs = scr[2:2 + NBUF]
    gse = scr[2 + NBUF:2 + 2 * NBUF]
    sse = scr[2 + 2 * NBUF:2 + 3 * NBUF]
    xq_sh = scr[2 + 3 * NBUF]
    acc_sh = scr[3 + 3 * NBUF]
    c = lax.axis_index("c")
    s = lax.axis_index("s")
    zb = bufs[0]

    def qpass(xn, outr):
        pltpu.sync_copy(xn.at[pl.ds(s * SEG, SEG)],
                        xq_sh.at[pl.ds(s * SEG, SEG)])

        def zfill(i, carry):
            zb[i // 2, pl.ds((i % 2) * 16, 16)] = jnp.zeros((16,), jnp.float32)
            return carry
        lax.fori_loop(0, LANE * (DQ // 16), zfill, 0)

        def zc(j, carry):
            pltpu.sync_copy(zb, acc_sh.at[pl.ds(s * SEG + j * LANE, LANE)])
            return carry
        lax.fori_loop(0, SEG // LANE, zc, 0)
        plsc.subcore_barrier()

        for b in range(NBUF):
            pltpu.async_copy(xq_sh.at[sidx.at[b]], bufs[b], gse[b])

        def rnd(k, carry):
            for b in range(NBUF):
                j = k * NBUF + b
                pltpu.make_async_copy(xq_sh.at[sidx.at[j]], bufs[b],
                                      gse[b]).wait()
                pltpu.async_copy(bufs[b], acc_sh.at[didx.at[j]], sse[b],
                                 add=True)
            for b in range(NBUF):
                j = k * NBUF + b

                @pl.when(k < NRND - 1)
                def _(b=b, j=j):
                    pltpu.make_async_copy(bufs[b], acc_sh.at[didx.at[j]],
                                          sse[b]).wait()
                    pltpu.async_copy(xq_sh.at[sidx.at[j + NBUF]], bufs[b],
                                     gse[b])
            return carry
        lax.fori_loop(0, NRND, rnd, 0)
        for b in range(NBUF):
            pltpu.make_async_copy(bufs[b],
                                  acc_sh.at[didx.at[RPT - NBUF + b]],
                                  sse[b]).wait()
        plsc.subcore_barrier()

        def wc(j, carry):
            pltpu.sync_copy(acc_sh.at[pl.ds(s * SEG + j * LANE, LANE)],
                            outr.at[pl.ds(s * SEG + j * LANE, LANE)])
            return carry
        lax.fori_loop(0, SEG // LANE, wc, 0)

    def runboth(srcr, dstr, m):
        pltpu.sync_copy(srcr.at[pl.ds(s * RPT, RPT)], sidx)
        pltpu.sync_copy(dstr.at[pl.ds(s * RPT, RPT)], didx)
        for q in range(NQ):
            qpass(xnq[m][q], outq[m][q])

    @pl.when(c == 0)
    def _():
        runboth(src1, dst1, 0)

    @pl.when(c == 1)
    def _():
        runboth(src2, dst2, 1)


_agg_kernel = functools.partial(
    pl.kernel, _agg_body, mesh=_mesh,
    out_type=[jax.ShapeDtypeStruct((NPAD, DQ), jnp.float32)] * (2 * NQ),
    scratch_types=(
        [pltpu.VMEM((RPT, LANE), jnp.int32) for _ in range(2)]
        + [pltpu.VMEM((LANE, DQ), jnp.float32) for _ in range(NBUF)]
        + [pltpu.SemaphoreType.DMA for _ in range(2 * NBUF)]
        + [pltpu.VMEM_SHARED((NPAD, DQ), jnp.float32) for _ in range(2)]
    ),
    compiler_params=pltpu.CompilerParams(use_tc_tiling_on_sc=False),
)()


BN = 1024


def _scale_body(x_ref, dego_ref, *outs):
    sc = lax.rsqrt(jnp.maximum(dego_ref[...], 1.0))
    xv = x_ref[...]
    xn1 = xv * sc[:, 0:1]
    xn2 = xv * sc[:, 1:2]
    for q in range(NQ):
        outs[q][...] = xn1[:, q * DQ:(q + 1) * DQ]
        outs[NQ + q][...] = xn2[:, q * DQ:(q + 1) * DQ]


def _scale_call(x_pad, dego):
    return pl.pallas_call(
        _scale_body,
        grid=(NPAD // BN,),
        in_specs=[
            pl.BlockSpec((BN, D), lambda i: (i, 0)),
            pl.BlockSpec((BN, 2), lambda i: (i, 0)),
        ],
        out_specs=[pl.BlockSpec((BN, DQ), lambda i: (i, 0))] * (2 * NQ),
        out_shape=[jax.ShapeDtypeStruct((NPAD, DQ), jnp.float32)] * (2 * NQ),
    )(x_pad, dego)


BC = 1000
NB = N // BC


def _post_body(*refs):
    aq = refs[0:2 * NQ]
    degi_ref, wg_ref, bg_ref, w1_ref, b1_ref, w2_ref = refs[2 * NQ:2 * NQ + 6]
    z_ref, part_ref = refs[2 * NQ + 6], refs[2 * NQ + 7]
    si = lax.rsqrt(jnp.maximum(degi_ref[...], 1.0))
    wg = wg_ref[...]
    bg = bg_ref[...]
    a1 = jnp.concatenate([aq[q][...] for q in range(NQ)], axis=1)
    a2 = jnp.concatenate([aq[NQ + q][...] for q in range(NQ)], axis=1)
    e1 = jnp.maximum(jnp.dot(a1 * si[:, 0:1], wg,
                             preferred_element_type=jnp.float32) + bg, 0.0)
    e2 = jnp.maximum(jnp.dot(a2 * si[:, 1:2], wg,
                             preferred_element_type=jnp.float32) + bg, 0.0)
    z_ref[...] = jnp.stack([e1, e2], axis=1)
    w1 = w1_ref[...]
    b1 = b1_ref[...]
    w2 = w2_ref[...]
    p1 = jnp.sum(jnp.dot(jnp.tanh(jnp.dot(e1, w1,
                                          preferred_element_type=jnp.float32)
                                  + b1), w2,
                         preferred_element_type=jnp.float32))
    p2 = jnp.sum(jnp.dot(jnp.tanh(jnp.dot(e2, w1,
                                          preferred_element_type=jnp.float32)
                                  + b1), w2,
                         preferred_element_type=jnp.float32))
    part_ref[...] = jnp.stack([p1, p2]).reshape(1, 1, 2)


def _post_call(aggq, degi, Wg, bg2, W1, b12, W2):
    return pl.pallas_call(
        _post_body,
        grid=(NB,),
        in_specs=[pl.BlockSpec((BC, DQ), lambda i: (i, 0))] * (2 * NQ) + [
            pl.BlockSpec((BC, 2), lambda i: (i, 0)),
            pl.BlockSpec((D, D), lambda i: (0, 0)),
            pl.BlockSpec((1, D), lambda i: (0, 0)),
            pl.BlockSpec((D, HID), lambda i: (0, 0)),
            pl.BlockSpec((1, HID), lambda i: (0, 0)),
            pl.BlockSpec((HID, 1), lambda i: (0, 0)),
        ],
        out_specs=[
            pl.BlockSpec((BC, 2, D), lambda i: (i, 0, 0)),
            pl.BlockSpec((1, 1, 2), lambda i: (i, 0, 0)),
        ],
        out_shape=[
            jax.ShapeDtypeStruct((N, 2, D), jnp.float32),
            jax.ShapeDtypeStruct((NB, 1, 2), jnp.float32),
        ],
    )(*aggq, degi, Wg, bg2, W1, b12, W2)


def _comb_body(part_ref, z_ref, out_ref):
    w = jnp.sum(part_ref[...], axis=0) * (1.0 / N)
    m = jnp.max(w)
    ew = jnp.exp(w - m)
    beta = ew / jnp.sum(ew)
    zz = z_ref[...]
    out_ref[...] = (zz[:, 0, :] * beta[0:1, 0:1]
                    + zz[:, 1, :] * beta[0:1, 1:2])


def _comb_call(part, z):
    return pl.pallas_call(
        _comb_body,
        grid=(NB,),
        in_specs=[
            pl.BlockSpec((NB, 1, 2), lambda i: (0, 0, 0)),
            pl.BlockSpec((BC, 2, D), lambda i: (i, 0, 0)),
        ],
        out_specs=pl.BlockSpec((BC, D), lambda i: (i, 0)),
        out_shape=jax.ShapeDtypeStruct((N, D), jnp.float32),
    )(part, z)


def _pad_idx(a):
    a = jnp.pad(a, (0, EPAD - E), constant_values=PADIDX)
    return a.reshape(ROWS, LANE)


def kernel(x, edge_index1, edge_index2, Wg, bg, W1, b1, W2):
    src1 = _pad_idx(edge_index1[0])
    dst1 = _pad_idx(edge_index1[1])
    src2 = _pad_idx(edge_index2[0])
    dst2 = _pad_idx(edge_index2[1])

    do1, di1, do2, di2 = _deg_kernel(src1, dst1, src2, dst2)
    dego = jnp.stack([do1, do2], axis=1)
    degi = jnp.stack([di1, di2], axis=1)

    x_pad = jnp.pad(x, ((0, NPAD - N), (0, 0)))
    xnq = _scale_call(x_pad, dego)

    aggq = _agg_kernel(*xnq, src1, dst1, src2, dst2)

    z, part = _post_call(aggq, degi, Wg, bg.reshape(1, D),
                         W1, b1.reshape(1, HID), W2)
    out = _comb_call(part, z)
    return (out, z)


# speedup vs baseline: 1.6429x; 1.0208x over previous
import functools

import jax
import jax.numpy as jnp
from jax import lax
from jax.experimental import pallas as pl
from jax.experimental.pallas import tpu as pltpu
from jax.experimental.pallas import tpu_sc as plsc

N = 10000
E = 320000
D = 128
HID = 32

NS = 16
LANE = 128
NPAD = 10240
SEG = NPAD // NS
PADIDX = NPAD - 1
ROWS = 2560
EPAD = ROWS * LANE
RPT = ROWS // NS

_mesh = plsc.VectorSubcoreMesh(core_axis_name="c", subcore_axis_name="s")


DEGQ = 8


def _deg_body(src1, dst1, src2, dst2, o1, i1, o2, i2,
              sidx, didx, ones, zv, se0, se1, dego_sh, degi_sh):
    c = lax.axis_index("c")
    s = lax.axis_index("s")

    def fill(ref, val, n):
        def b(i, carry):
            ref[pl.ds(i * 16, 16)] = jnp.full((16,), val, jnp.float32)
            return carry
        lax.fori_loop(0, n // 16, b, 0)

    fill(ones, 1.0, LANE)
    fill(zv, 0.0, SEG)
    pltpu.sync_copy(zv, dego_sh.at[pl.ds(s * SEG, SEG)])
    pltpu.sync_copy(zv, degi_sh.at[pl.ds(s * SEG, SEG)])
    plsc.subcore_barrier()

    def run(srcr, dstr, outo, outi):
        pltpu.sync_copy(srcr.at[pl.ds(s * RPT, RPT)], sidx)
        pltpu.sync_copy(dstr.at[pl.ds(s * RPT, RPT)], didx)

        def b(j, carry):
            @pl.when(j >= DEGQ)
            def _():
                pltpu.make_async_copy(ones, dego_sh.at[sidx.at[j - DEGQ]],
                                      se0).wait()
                pltpu.make_async_copy(ones, degi_sh.at[didx.at[j - DEGQ]],
                                      se1).wait()
            pltpu.async_copy(ones, dego_sh.at[sidx.at[j]], se0, add=True)
            pltpu.async_copy(ones, degi_sh.at[didx.at[j]], se1, add=True)
            return carry
        lax.fori_loop(0, RPT, b, 0)
        for j in range(RPT - DEGQ, RPT):
            pltpu.make_async_copy(ones, dego_sh.at[sidx.at[j]], se0).wait()
            pltpu.make_async_copy(ones, degi_sh.at[didx.at[j]], se1).wait()
        plsc.subcore_barrier()
        pltpu.sync_copy(dego_sh.at[pl.ds(s * SEG, SEG)],
                        outo.at[pl.ds(s * SEG, SEG)])
        pltpu.sync_copy(degi_sh.at[pl.ds(s * SEG, SEG)],
                        outi.at[pl.ds(s * SEG, SEG)])

    @pl.when(c == 0)
    def _():
        run(src1, dst1, o1, i1)

    @pl.when(c == 1)
    def _():
        run(src2, dst2, o2, i2)


_deg_kernel = functools.partial(
    pl.kernel, _deg_body, mesh=_mesh,
    out_type=[jax.ShapeDtypeStruct((NPAD,), jnp.float32)] * 4,
    scratch_types=[
        pltpu.VMEM((RPT, LANE), jnp.int32),
        pltpu.VMEM((RPT, LANE), jnp.int32),
        pltpu.VMEM((LANE,), jnp.float32),
        pltpu.VMEM((SEG,), jnp.float32),
        pltpu.SemaphoreType.DMA,
        pltpu.SemaphoreType.DMA,
        pltpu.VMEM_SHARED((NPAD,), jnp.float32),
        pltpu.VMEM_SHARED((NPAD,), jnp.float32),
    ],
)()


NQ = 4
DQ = D // NQ
NBUF = 10
NRND = RPT // NBUF


def _agg_body(*refs):
    xnq = (refs[0:NQ], refs[NQ:2 * NQ])
    src1, dst1, src2, dst2 = refs[2 * NQ:2 * NQ + 4]
    outq = (refs[2 * NQ + 4:3 * NQ + 4], refs[3 * NQ + 4:4 * NQ + 4])
    scr = refs[4 * NQ + 4:]
    sidx, didx = scr[0], scr[1]
    bufs = scr[2:2 + NBUF]
    gse = scr[2 + NBUF:2 + 2 * NBUF]
    sse = scr[2 + 2 * NBUF:2 + 3 * NBUF]
    xq_sh = scr[2 + 3 * NBUF]
    acc_sh = scr[3 + 3 * NBUF]
    c = lax.axis_index("c")
    s = lax.axis_index("s")
    zb = bufs[0]

    def qpass(xn, outr):
        pltpu.sync_copy(xn.at[pl.ds(s * SEG, SEG)],
                        xq_sh.at[pl.ds(s * SEG, SEG)])

        def zfill(i, carry):
            zb[i // 2, pl.ds((i % 2) * 16, 16)] = jnp.zeros((16,), jnp.float32)
            return carry
        lax.fori_loop(0, LANE * (DQ // 16), zfill, 0)

        def zc(j, carry):
            pltpu.sync_copy(zb, acc_sh.at[pl.ds(s * SEG + j * LANE, LANE)])
            return carry
        lax.fori_loop(0, SEG // LANE, zc, 0)
        plsc.subcore_barrier()

        for b in range(NBUF):
            pltpu.async_copy(xq_sh.at[sidx.at[b]], bufs[b], gse[b])

        def rnd(k, carry):
            for b in range(NBUF):
                j = k * NBUF + b
                pltpu.make_async_copy(xq_sh.at[sidx.at[j]], bufs[b],
                                      gse[b]).wait()
                pltpu.async_copy(bufs[b], acc_sh.at[didx.at[j]], sse[b],
                                 add=True)
            for b in range(NBUF):
                j = k * NBUF + b

                @pl.when(k < NRND - 1)
                def _(b=b, j=j):
                    pltpu.make_async_copy(bufs[b], acc_sh.at[didx.at[j]],
                                          sse[b]).wait()
                    pltpu.async_copy(xq_sh.at[sidx.at[j + NBUF]], bufs[b],
                                     gse[b])
            return carry
        lax.fori_loop(0, NRND, rnd, 0)
        for b in range(NBUF):
            pltpu.make_async_copy(bufs[b],
                                  acc_sh.at[didx.at[RPT - NBUF + b]],
                                  sse[b]).wait()
        plsc.subcore_barrier()

        def wc(j, carry):
            pltpu.sync_copy(acc_sh.at[pl.ds(s * SEG + j * LANE, LANE)],
                            outr.at[pl.ds(s * SEG + j * LANE, LANE)])
            return carry
        lax.fori_loop(0, SEG // LANE, wc, 0)

    def runboth(srcr, dstr, m):
        pltpu.sync_copy(srcr.at[pl.ds(s * RPT, RPT)], sidx)
        pltpu.sync_copy(dstr.at[pl.ds(s * RPT, RPT)], didx)
        for q in range(NQ):
            qpass(xnq[m][q], outq[m][q])

    @pl.when(c == 0)
    def _():
        runboth(src1, dst1, 0)

    @pl.when(c == 1)
    def _():
        runboth(src2, dst2, 1)


_agg_kernel = functools.partial(
    pl.kernel, _agg_body, mesh=_mesh,
    out_type=[jax.ShapeDtypeStruct((NPAD, DQ), jnp.float32)] * (2 * NQ),
    scratch_types=(
        [pltpu.VMEM((RPT, LANE), jnp.int32) for _ in range(2)]
        + [pltpu.VMEM((LANE, DQ), jnp.float32) for _ in range(NBUF)]
        + [pltpu.SemaphoreType.DMA for _ in range(2 * NBUF)]
        + [pltpu.VMEM_SHARED((NPAD, DQ), jnp.float32) for _ in range(2)]
    ),
    compiler_params=pltpu.CompilerParams(use_tc_tiling_on_sc=False),
)()


BN = 1024


def _scale_body(x_ref, dego_ref, *outs):
    sc = lax.rsqrt(jnp.maximum(dego_ref[...], 1.0))
    xv = x_ref[...]
    xn1 = xv * sc[:, 0:1]
    xn2 = xv * sc[:, 1:2]
    for q in range(NQ):
        outs[q][...] = xn1[:, q * DQ:(q + 1) * DQ]
        outs[NQ + q][...] = xn2[:, q * DQ:(q + 1) * DQ]


def _scale_call(x_pad, dego):
    return pl.pallas_call(
        _scale_body,
        grid=(NPAD // BN,),
        in_specs=[
            pl.BlockSpec((BN, D), lambda i: (i, 0)),
            pl.BlockSpec((BN, 2), lambda i: (i, 0)),
        ],
        out_specs=[pl.BlockSpec((BN, DQ), lambda i: (i, 0))] * (2 * NQ),
        out_shape=[jax.ShapeDtypeStruct((NPAD, DQ), jnp.float32)] * (2 * NQ),
    )(x_pad, dego)


BC = 1000
NB = N // BC


def _post_body(*refs):
    aq = refs[0:2 * NQ]
    degi_ref, wg_ref, bg_ref, w1_ref, b1_ref, w2_ref = refs[2 * NQ:2 * NQ + 6]
    z_ref, part_ref = refs[2 * NQ + 6], refs[2 * NQ + 7]
    si = lax.rsqrt(jnp.maximum(degi_ref[...], 1.0))
    wg = wg_ref[...]
    bg = bg_ref[...]
    a1 = jnp.concatenate([aq[q][...] for q in range(NQ)], axis=1)
    a2 = jnp.concatenate([aq[NQ + q][...] for q in range(NQ)], axis=1)
    e1 = jnp.maximum(jnp.dot(a1 * si[:, 0:1], wg,
                             preferred_element_type=jnp.float32) + bg, 0.0)
    e2 = jnp.maximum(jnp.dot(a2 * si[:, 1:2], wg,
                             preferred_element_type=jnp.float32) + bg, 0.0)
    z_ref[...] = jnp.stack([e1, e2], axis=1)
    w1 = w1_ref[...]
    b1 = b1_ref[...]
    w2 = w2_ref[...]
    p1 = jnp.sum(jnp.dot(jnp.tanh(jnp.dot(e1, w1,
                                          preferred_element_type=jnp.float32)
                                  + b1), w2,
                         preferred_element_type=jnp.float32))
    p2 = jnp.sum(jnp.dot(jnp.tanh(jnp.dot(e2, w1,
                                          preferred_element_type=jnp.float32)
                                  + b1), w2,
                         preferred_element_type=jnp.float32))
    part_ref[...] = jnp.stack([p1, p2]).reshape(1, 1, 2)


def _post_call(aggq, degi, Wg, bg2, W1, b12, W2):
    return pl.pallas_call(
        _post_body,
        grid=(NB,),
        in_specs=[pl.BlockSpec((BC, DQ), lambda i: (i, 0))] * (2 * NQ) + [
            pl.BlockSpec((BC, 2), lambda i: (i, 0)),
            pl.BlockSpec((D, D), lambda i: (0, 0)),
            pl.BlockSpec((1, D), lambda i: (0, 0)),
            pl.BlockSpec((D, HID), lambda i: (0, 0)),
            pl.BlockSpec((1, HID), lambda i: (0, 0)),
            pl.BlockSpec((HID, 1), lambda i: (0, 0)),
        ],
        out_specs=[
            pl.BlockSpec((BC, 2, D), lambda i: (i, 0, 0)),
            pl.BlockSpec((1, 1, 2), lambda i: (i, 0, 0)),
        ],
        out_shape=[
            jax.ShapeDtypeStruct((N, 2, D), jnp.float32),
            jax.ShapeDtypeStruct((NB, 1, 2), jnp.float32),
        ],
    )(*aggq, degi, Wg, bg2, W1, b12, W2)


def _comb_body(part_ref, z_ref, out_ref):
    w = jnp.sum(part_ref[...], axis=0) * (1.0 / N)
    m = jnp.max(w)
    ew = jnp.exp(w - m)
    beta = ew / jnp.sum(ew)
    zz = z_ref[...]
    out_ref[...] = (zz[:, 0, :] * beta[0:1, 0:1]
                    + zz[:, 1, :] * beta[0:1, 1:2])


def _comb_call(part, z):
    return pl.pallas_call(
        _comb_body,
        grid=(NB,),
        in_specs=[
            pl.BlockSpec((NB, 1, 2), lambda i: (0, 0, 0)),
            pl.BlockSpec((BC, 2, D), lambda i: (i, 0, 0)),
        ],
        out_specs=pl.BlockSpec((BC, D), lambda i: (i, 0)),
        out_shape=jax.ShapeDtypeStruct((N, D), jnp.float32),
    )(part, z)


def _pad_idx(a):
    a = jnp.pad(a, (0, EPAD - E), constant_values=PADIDX)
    return a.reshape(ROWS, LANE)


def kernel(x, edge_index1, edge_index2, Wg, bg, W1, b1, W2):
    src1 = _pad_idx(edge_index1[0])
    dst1 = _pad_idx(edge_index1[1])
    src2 = _pad_idx(edge_index2[0])
    dst2 = _pad_idx(edge_index2[1])

    do1, di1, do2, di2 = _deg_kernel(src1, dst1, src2, dst2)
    dego = jnp.stack([do1, do2], axis=1)
    degi = jnp.stack([di1, di2], axis=1)

    x_pad = jnp.pad(x, ((0, NPAD - N), (0, 0)))
    xnq = _scale_call(x_pad, dego)

    aggq = _agg_kernel(*xnq, src1, dst1, src2, dst2)

    z, part = _post_call(aggq, degi, Wg, bg.reshape(1, D),
                         W1, b1.reshape(1, HID), W2)
    out = _comb_call(part, z)
    return (out, z)
